# baseline (speedup 1.0000x reference)
import functools

import jax
import jax.numpy as jnp
from jax.experimental import pallas as pl
from jax.experimental.pallas import tpu as pltpu

_K_TOP = 32


def _encode_kernel(x_ref, w_ref, b_ref, out_ref):
    acc = jax.lax.dot_general(
        x_ref[...], w_ref[...],
        dimension_numbers=(((1,), (1,)), ((), ())),
        preferred_element_type=jnp.float32,
        precision=jax.lax.Precision.DEFAULT,
    )
    out_ref[...] = acc + b_ref[...]


def _threshold_search(P, t_ref, lo_ref, hi_ref, cl_ref, ch_ref,
                      k, iters, interp_iters):
    kf = jnp.float32(k)
    rows = P.shape[0]

    def count(t):
        return jnp.sum((P >= t).astype(jnp.float32), axis=1, keepdims=True)

    rmax = jnp.max(P, axis=1, keepdims=True)
    rmin = jnp.min(P, axis=1, keepdims=True)
    c_max = count(rmax)
    deg = c_max >= kf
    lo_ref[...] = jnp.where(deg, rmax, rmin)
    hi_ref[...] = rmax
    cl_ref[...] = jnp.where(deg, c_max, jnp.float32(P.shape[1]))
    ch_ref[...] = c_max

    def cond(st):
        i, ndone = st
        return jnp.logical_and(i < iters, ndone < rows)

    def body(st):
        i, _ = st
        lo = lo_ref[...]
        hi = hi_ref[...]
        llo = jnp.log(jnp.maximum(cl_ref[...], 1.0))
        lhi = jnp.log(jnp.maximum(ch_ref[...], 0.5))
        lk = jnp.log(kf)
        frac = (llo - lk) / jnp.maximum(llo - lhi, jnp.float32(1e-6))
        frac = jnp.clip(frac, 0.08, 0.92)
        frac = jnp.where(i < interp_iters, frac, jnp.float32(0.5))
        mid = lo + frac * (hi - lo)
        c = count(mid)
        ge = c >= kf
        le = c <= kf
        lo_ref[...] = jnp.where(ge, mid, lo)
        hi_ref[...] = jnp.where(le, mid, hi)
        cl_ref[...] = jnp.where(ge, c, cl_ref[...])
        ch_ref[...] = jnp.where(le, c, ch_ref[...])
        done = jnp.logical_or(c == kf,
                              jnp.logical_or(mid == lo, mid == hi))
        return i + 1, jnp.sum(done.astype(jnp.float32))

    jax.lax.while_loop(cond, body, (jnp.int32(0), jnp.float32(0.0)))
    t_ref[...] = lo_ref[...]


def _select_decode_kernel(p_ref, w_ref, acts_ref, recon_ref,
                          t_ref, lo_ref, hi_ref, cl_ref, ch_ref,
                          *, k, iters, interp_iters, rd, fd):
    r = pl.program_id(0)
    s = pl.program_id(1)

    @pl.when(s == 0)
    def _():
        _threshold_search(p_ref[...], t_ref, lo_ref, hi_ref, cl_ref,
                          ch_ref, k, iters, interp_iters)

    @pl.when(s > 0)
    def _():
        j = s - 1
        tile = p_ref[:, pl.ds(j * fd, fd)]
        t = t_ref[...]
        acts = jnp.where(tile >= t, jnp.maximum(tile, 0.0), 0.0)
        acts_ref[...] = acts
        contrib = jax.lax.dot_general(
            acts, w_ref[...],
            dimension_numbers=(((1,), (1,)), ((), ())),
            preferred_element_type=jnp.float32,
            precision=jax.lax.Precision.DEFAULT,
        )
        rs = pl.ds(r * rd, rd)

        @pl.when(s == 1)
        def _():
            recon_ref[rs, :] = contrib

        @pl.when(s > 1)
        def _():
            recon_ref[rs, :] = recon_ref[rs, :] + contrib


def kernel(x, W_enc, b_enc, W_dec):
    n, d = x.shape
    nf = W_enc.shape[0]
    f32 = jnp.float32

    fj = min(2048, nf)
    nj1 = nf // fj
    re = min(1024, n)
    nre = n // re
    b2 = b_enc.reshape(1, nf).astype(f32)
    pre = pl.pallas_call(
        _encode_kernel,
        grid=(nj1, nre),
        in_specs=[
            pl.BlockSpec((re, d), lambda j, r: (r, 0)),
            pl.BlockSpec((fj, d), lambda j, r: (j, 0)),
            pl.BlockSpec((1, fj), lambda j, r: (0, j)),
        ],
        out_specs=pl.BlockSpec((re, fj), lambda j, r: (r, j)),
        out_shape=jax.ShapeDtypeStruct((n, nf), f32),
    )(x.astype(f32), W_enc.astype(f32), b2)

    rd = min(64, n)
    nrd = n // rd
    fd = min(2048, nf)
    nj2 = nf // fd
    acts, recon = pl.pallas_call(
        functools.partial(_select_decode_kernel, k=_K_TOP, iters=46,
                          interp_iters=14, rd=rd, fd=fd),
        grid=(nrd, nj2 + 1),
        in_specs=[
            pl.BlockSpec((rd, nf), lambda r, s: (r, 0)),
            pl.BlockSpec((d, fd),
                         lambda r, s: (0, jnp.maximum(s - 1, 0))),
        ],
        out_specs=[
            pl.BlockSpec((rd, fd),
                         lambda r, s: (r, jnp.maximum(s - 1, 0))),
            pl.BlockSpec((n, d), lambda r, s: (0, 0)),
        ],
        out_shape=[
            jax.ShapeDtypeStruct((n, nf), f32),
            jax.ShapeDtypeStruct((n, d), f32),
        ],
        scratch_shapes=[
            pltpu.VMEM((rd, 1), f32),
            pltpu.VMEM((rd, 1), f32),
            pltpu.VMEM((rd, 1), f32),
            pltpu.VMEM((rd, 1), f32),
            pltpu.VMEM((rd, 1), f32),
        ],
    )(pre, W_dec.astype(f32))

    return recon, acts


# speedup vs baseline: 1.8326x; 1.8326x over previous
import functools

import jax
import jax.numpy as jnp
from jax.experimental import pallas as pl
from jax.experimental.pallas import tpu as pltpu

_K_TOP = 32


def _encode_kernel(x_ref, w_ref, b_ref, out_ref):
    acc = jax.lax.dot_general(
        x_ref[...], w_ref[...],
        dimension_numbers=(((1,), (1,)), ((), ())),
        preferred_element_type=jnp.float32,
        precision=jax.lax.Precision.DEFAULT,
    )
    out_ref[...] = acc + b_ref[...]


def _threshold_kernel(p_ref, t_ref, lo_ref, hi_ref, cl_ref, ch_ref,
                      *, k, iters, interp_iters, iters_b, extract_margin):
    P = p_ref[...]
    kf = jnp.float32(k)
    rows = P.shape[0]

    def count2(t1, t2):
        ones = jnp.int32(1)
        hi16 = jnp.int32(65536)
        zero = jnp.int32(0)
        packed = jnp.sum(
            jnp.where(P >= t1, ones, zero)
            + jnp.where(P >= t2, hi16, zero),
            axis=1, keepdims=True)
        c1 = (packed & jnp.int32(0xFFFF)).astype(jnp.float32)
        c2 = (packed >> 16).astype(jnp.float32)
        return c1, c2

    rmax = jnp.max(P, axis=1, keepdims=True)
    rmin = jnp.min(P, axis=1, keepdims=True)
    lo_ref[...] = rmin
    hi_ref[...] = rmax
    cl_ref[...] = jnp.full_like(rmax, jnp.float32(P.shape[1]))
    ch_ref[...] = jnp.full_like(rmax, jnp.float32(1.0))

    def cond(st):
        i, ndone = st
        return jnp.logical_and(i < iters, ndone < rows)

    def body(st):
        i, _ = st
        lo = lo_ref[...]
        hi = hi_ref[...]
        cl = cl_ref[...]
        ch = ch_ref[...]
        llo = jnp.log(jnp.maximum(cl, 1.0))
        lhi = jnp.log(jnp.maximum(ch, 0.5))
        lk = jnp.log(kf)
        frac = (llo - lk) / jnp.maximum(llo - lhi, jnp.float32(1e-6))
        interp = i < interp_iters
        f1 = jnp.where(interp, jnp.clip(frac - 0.10, 0.03, 0.90),
                       jnp.float32(1.0 / 3.0))
        f2 = jnp.where(interp, jnp.clip(frac + 0.10, 0.06, 0.97),
                       jnp.float32(2.0 / 3.0))
        w = hi - lo
        m1 = lo + f1 * w
        m2 = lo + f2 * w
        c1, c2 = count2(m1, m2)
        hit = jnp.logical_or(c1 == kf, c2 == kf)
        tsel = jnp.where(c1 == kf, m1, m2)
        new_lo = jnp.where(
            hit, tsel,
            jnp.where(c2 >= kf, m2, jnp.where(c1 >= kf, m1, lo)))
        new_hi = jnp.where(
            hit, tsel,
            jnp.where(c1 <= kf, m1, jnp.where(c2 <= kf, m2, hi)))
        cl_ref[...] = jnp.where(c2 >= kf, c2,
                                jnp.where(c1 >= kf, c1, cl))
        ch_ref[...] = jnp.where(c1 <= kf, c1,
                                jnp.where(c2 <= kf, c2, ch))
        lo_ref[...] = new_lo
        hi_ref[...] = new_hi
        done = jnp.logical_or(
            jnp.logical_or(hit, m1 <= lo),
            ch_ref[...] >= kf - jnp.float32(extract_margin))
        return i + 1, jnp.sum(done.astype(jnp.float32))

    jax.lax.while_loop(cond, body, (jnp.int32(0), jnp.float32(0.0)))

    def cond_b(st):
        i, nrem = st
        return jnp.logical_and(i < iters_b, nrem > 0)

    def body_b(st):
        i, _ = st
        hi = hi_ref[...]
        lo = lo_ref[...]
        ch = ch_ref[...]
        active = jnp.logical_and(ch < kf, hi > lo)
        m = jnp.max(
            jnp.where(jnp.logical_and(P < hi, active), P,
                      jnp.float32(-jnp.inf)),
            axis=1, keepdims=True)
        new_hi = jnp.where(active, m, hi)
        new_ch = jnp.where(active, ch + 1.0, ch)
        hi_ref[...] = new_hi
        ch_ref[...] = new_ch
        rem = jnp.logical_and(new_ch < kf, new_hi > lo)
        return i + 1, jnp.sum(rem.astype(jnp.float32))

    jax.lax.while_loop(cond_b, body_b, (jnp.int32(0), jnp.float32(1.0)))
    t_ref[...] = jnp.where(ch_ref[...] == kf, hi_ref[...], lo_ref[...])


def _decode_kernel(p_ref, w_ref, t_ref, acts_ref, recon_ref, *, rd):
    j = pl.program_id(0)
    r = pl.program_id(1)
    tile = p_ref[...]
    t = t_ref[...]
    acts = jnp.where(tile >= t, jnp.maximum(tile, 0.0), 0.0)
    acts_ref[...] = acts
    contrib = jax.lax.dot_general(
        acts, w_ref[...],
        dimension_numbers=(((1,), (1,)), ((), ())),
        preferred_element_type=jnp.float32,
        precision=jax.lax.Precision.DEFAULT,
    )
    rs = pl.ds(r * rd, rd)

    @pl.when(j == 0)
    def _():
        recon_ref[rs, :] = contrib

    @pl.when(j > 0)
    def _():
        recon_ref[rs, :] = recon_ref[rs, :] + contrib


def kernel(x, W_enc, b_enc, W_dec):
    n, d = x.shape
    nf = W_enc.shape[0]
    f32 = jnp.float32

    fj = min(2048, nf)
    nj1 = nf // fj
    re = min(1024, n)
    nre = n // re
    b2 = b_enc.reshape(1, nf).astype(f32)
    pre = pl.pallas_call(
        _encode_kernel,
        grid=(nj1, nre),
        in_specs=[
            pl.BlockSpec((re, d), lambda j, r: (r, 0)),
            pl.BlockSpec((fj, d), lambda j, r: (j, 0)),
            pl.BlockSpec((1, fj), lambda j, r: (0, j)),
        ],
        out_specs=pl.BlockSpec((re, fj), lambda j, r: (r, j)),
        out_shape=jax.ShapeDtypeStruct((n, nf), f32),
    )(x.astype(f32), W_enc.astype(f32), b2)

    rt = min(128, n)
    nrt = n // rt
    thr = pl.pallas_call(
        functools.partial(_threshold_kernel, k=_K_TOP, iters=12,
                          interp_iters=10, iters_b=34,
                          extract_margin=8),
        grid=(nrt,),
        in_specs=[pl.BlockSpec((rt, nf), lambda r: (r, 0))],
        out_specs=pl.BlockSpec((rt, 1), lambda r: (r, 0)),
        out_shape=jax.ShapeDtypeStruct((n, 1), f32),
        scratch_shapes=[
            pltpu.VMEM((rt, 1), f32),
            pltpu.VMEM((rt, 1), f32),
            pltpu.VMEM((rt, 1), f32),
            pltpu.VMEM((rt, 1), f32),
        ],
    )(pre)

    rd = min(256, n)
    nrd = n // rd
    fd = min(2048, nf)
    nj2 = nf // fd
    acts, recon = pl.pallas_call(
        functools.partial(_decode_kernel, rd=rd),
        grid=(nj2, nrd),
        in_specs=[
            pl.BlockSpec((rd, fd), lambda j, r: (r, j)),
            pl.BlockSpec((d, fd), lambda j, r: (0, j)),
            pl.BlockSpec((rd, 1), lambda j, r: (r, 0)),
        ],
        out_specs=[
            pl.BlockSpec((rd, fd), lambda j, r: (r, j)),
            pl.BlockSpec((n, d), lambda j, r: (0, 0)),
        ],
        out_shape=[
            jax.ShapeDtypeStruct((n, nf), f32),
            jax.ShapeDtypeStruct((n, d), f32),
        ],
    )(pre, W_dec.astype(f32), thr)

    return recon, acts


# speedup vs baseline: 1.9736x; 1.0770x over previous
import functools

import jax
import jax.numpy as jnp
from jax.experimental import pallas as pl
from jax.experimental.pallas import tpu as pltpu

_K_TOP = 32


def _encode_kernel(x_ref, w_ref, b_ref, out_ref):
    acc = jax.lax.dot_general(
        x_ref[...], w_ref[...],
        dimension_numbers=(((1,), (1,)), ((), ())),
        preferred_element_type=jnp.float32,
        precision=jax.lax.Precision.DEFAULT,
    )
    out_ref[...] = acc + b_ref[...]


def _threshold_kernel(p_ref, t_ref, lo_ref, hi_ref, cl_ref, ch_ref,
                      *, k, iters, interp_iters):
    P = p_ref[...]
    kf = jnp.float32(k)
    rows = P.shape[0]

    def count(t):
        return jnp.sum((P >= t).astype(jnp.float32), axis=1, keepdims=True)

    rmax = jnp.max(P, axis=1, keepdims=True)
    rmin = jnp.min(P, axis=1, keepdims=True)
    c_max = count(rmax)
    deg = c_max >= kf
    lo_ref[...] = jnp.where(deg, rmax, rmin)
    hi_ref[...] = rmax
    cl_ref[...] = jnp.where(deg, c_max, jnp.float32(P.shape[1]))
    ch_ref[...] = c_max

    def cond(st):
        i, ndone = st
        return jnp.logical_and(i < iters, ndone < rows)

    def body(st):
        i, _ = st
        lo = lo_ref[...]
        hi = hi_ref[...]
        llo = jnp.log(jnp.maximum(cl_ref[...], 1.0))
        lhi = jnp.log(jnp.maximum(ch_ref[...], 0.5))
        lk = jnp.log(kf)
        frac = (llo - lk) / jnp.maximum(llo - lhi, jnp.float32(1e-6))
        frac = jnp.clip(frac, 0.08, 0.92)
        frac = jnp.where(i < interp_iters, frac, jnp.float32(0.5))
        mid = lo + frac * (hi - lo)
        c = count(mid)
        ge = c >= kf
        le = c <= kf
        lo_ref[...] = jnp.where(ge, mid, lo)
        hi_ref[...] = jnp.where(le, mid, hi)
        cl_ref[...] = jnp.where(ge, c, cl_ref[...])
        ch_ref[...] = jnp.where(le, c, ch_ref[...])
        done = jnp.logical_or(c == kf,
                              jnp.logical_or(mid == lo, mid == hi))
        return i + 1, jnp.sum(done.astype(jnp.float32))

    jax.lax.while_loop(cond, body, (jnp.int32(0), jnp.float32(0.0)))
    t_ref[...] = lo_ref[...]


def _decode_kernel(p_ref, w_ref, t_ref, acts_ref, recon_ref, *, rd):
    j = pl.program_id(0)
    r = pl.program_id(1)
    tile = p_ref[...]
    t = t_ref[...]
    acts = jnp.where(tile >= t, jnp.maximum(tile, 0.0), 0.0)
    acts_ref[...] = acts
    contrib = jax.lax.dot_general(
        acts, w_ref[...],
        dimension_numbers=(((1,), (1,)), ((), ())),
        preferred_element_type=jnp.float32,
        precision=jax.lax.Precision.DEFAULT,
    )
    rs = pl.ds(r * rd, rd)

    @pl.when(j == 0)
    def _():
        recon_ref[rs, :] = contrib

    @pl.when(j > 0)
    def _():
        recon_ref[rs, :] = recon_ref[rs, :] + contrib


def kernel(x, W_enc, b_enc, W_dec):
    n, d = x.shape
    nf = W_enc.shape[0]
    f32 = jnp.float32

    fj = min(2048, nf)
    nj1 = nf // fj
    re = min(1024, n)
    nre = n // re
    b2 = b_enc.reshape(1, nf).astype(f32)
    pre = pl.pallas_call(
        _encode_kernel,
        grid=(nj1, nre),
        in_specs=[
            pl.BlockSpec((re, d), lambda j, r: (r, 0)),
            pl.BlockSpec((fj, d), lambda j, r: (j, 0)),
            pl.BlockSpec((1, fj), lambda j, r: (0, j)),
        ],
        out_specs=pl.BlockSpec((re, fj), lambda j, r: (r, j)),
        out_shape=jax.ShapeDtypeStruct((n, nf), f32),
    )(x.astype(f32), W_enc.astype(f32), b2)

    rt = min(64, n)
    nrt = n // rt
    thr = pl.pallas_call(
        functools.partial(_threshold_kernel, k=_K_TOP, iters=46,
                          interp_iters=14),
        grid=(nrt,),
        in_specs=[pl.BlockSpec((rt, nf), lambda r: (r, 0))],
        out_specs=pl.BlockSpec((rt, 1), lambda r: (r, 0)),
        out_shape=jax.ShapeDtypeStruct((n, 1), f32),
        scratch_shapes=[
            pltpu.VMEM((rt, 1), f32),
            pltpu.VMEM((rt, 1), f32),
            pltpu.VMEM((rt, 1), f32),
            pltpu.VMEM((rt, 1), f32),
        ],
    )(pre)

    rd = min(256, n)
    nrd = n // rd
    fd = min(4096, nf)
    nj2 = nf // fd
    acts, recon = pl.pallas_call(
        functools.partial(_decode_kernel, rd=rd),
        grid=(nj2, nrd),
        in_specs=[
            pl.BlockSpec((rd, fd), lambda j, r: (r, j)),
            pl.BlockSpec((d, fd), lambda j, r: (0, j)),
            pl.BlockSpec((rd, 1), lambda j, r: (r, 0)),
        ],
        out_specs=[
            pl.BlockSpec((rd, fd), lambda j, r: (r, j)),
            pl.BlockSpec((n, d), lambda j, r: (0, 0)),
        ],
        out_shape=[
            jax.ShapeDtypeStruct((n, nf), f32),
            jax.ShapeDtypeStruct((n, d), f32),
        ],
    )(pre, W_dec.astype(f32), thr)

    return recon, acts


# speedup vs baseline: 2.0678x; 1.0477x over previous
import functools

import jax
import jax.numpy as jnp
from jax.experimental import pallas as pl
from jax.experimental.pallas import tpu as pltpu

_K_TOP = 32


def _encode_kernel(x_ref, w_ref, b_ref, out_ref):
    acc = jax.lax.dot_general(
        x_ref[...], w_ref[...],
        dimension_numbers=(((1,), (1,)), ((), ())),
        preferred_element_type=jnp.float32,
        precision=jax.lax.Precision.DEFAULT,
    )
    out_ref[...] = acc + b_ref[...]


def _threshold_kernel(p_ref, t_ref, lo_ref, hi_ref, cl_ref, ch_ref,
                      *, k, iters, interp_iters):
    P = p_ref[...]
    kf = jnp.float32(k)
    rows = P.shape[0]

    def count(t):
        return jnp.sum((P >= t).astype(jnp.float32), axis=1, keepdims=True)

    rmax = jnp.max(P, axis=1, keepdims=True)
    rmin = jnp.min(P, axis=1, keepdims=True)
    lo_ref[...] = rmin
    hi_ref[...] = rmax
    cl_ref[...] = jnp.full_like(rmax, jnp.float32(P.shape[1]))
    ch_ref[...] = jnp.full_like(rmax, jnp.float32(1.0))

    def cond(st):
        i, ndone = st
        return jnp.logical_and(i < iters, ndone < rows)

    def body(st):
        i, _ = st
        lo = lo_ref[...]
        hi = hi_ref[...]
        llo = jnp.log(jnp.maximum(cl_ref[...], 1.0))
        lhi = jnp.log(jnp.maximum(ch_ref[...], 0.5))
        lk = jnp.log(kf)
        frac = (llo - lk) / jnp.maximum(llo - lhi, jnp.float32(1e-6))
        frac = jnp.clip(frac, 0.08, 0.92)
        frac = jnp.where(i < interp_iters, frac, jnp.float32(0.5))
        mid = lo + frac * (hi - lo)
        c = count(mid)
        ge = c >= kf
        le = c <= kf
        lo_ref[...] = jnp.where(ge, mid, lo)
        hi_ref[...] = jnp.where(le, mid, hi)
        cl_ref[...] = jnp.where(ge, c, cl_ref[...])
        ch_ref[...] = jnp.where(le, c, ch_ref[...])
        done = jnp.logical_or(c == kf,
                              jnp.logical_or(mid == lo, mid == hi))
        return i + 1, jnp.sum(done.astype(jnp.float32))

    jax.lax.while_loop(cond, body, (jnp.int32(0), jnp.float32(0.0)))
    t_ref[...] = lo_ref[...]


def _decode_kernel(p_ref, w_ref, t_ref, acts_ref, recon_ref, *, rd):
    j = pl.program_id(0)
    r = pl.program_id(1)
    tile = p_ref[...]
    t = t_ref[...]
    acts = jnp.where(tile >= t, jnp.maximum(tile, 0.0), 0.0)
    acts_ref[...] = acts
    contrib = jax.lax.dot_general(
        acts, w_ref[...],
        dimension_numbers=(((1,), (1,)), ((), ())),
        preferred_element_type=jnp.float32,
        precision=jax.lax.Precision.DEFAULT,
    )
    rs = pl.ds(r * rd, rd)

    @pl.when(j == 0)
    def _():
        recon_ref[rs, :] = contrib

    @pl.when(j > 0)
    def _():
        recon_ref[rs, :] = recon_ref[rs, :] + contrib


def kernel(x, W_enc, b_enc, W_dec):
    n, d = x.shape
    nf = W_enc.shape[0]
    f32 = jnp.float32

    fj = min(2048, nf)
    nj1 = nf // fj
    re = min(1024, n)
    nre = n // re
    b2 = b_enc.reshape(1, nf).astype(f32)
    pre = pl.pallas_call(
        _encode_kernel,
        grid=(nj1, nre),
        in_specs=[
            pl.BlockSpec((re, d), lambda j, r: (r, 0)),
            pl.BlockSpec((fj, d), lambda j, r: (j, 0)),
            pl.BlockSpec((1, fj), lambda j, r: (0, j)),
        ],
        out_specs=pl.BlockSpec((re, fj), lambda j, r: (r, j)),
        out_shape=jax.ShapeDtypeStruct((n, nf), f32),
    )(x.astype(f32), W_enc.astype(f32), b2)

    rt = min(128, n)
    nrt = n // rt
    thr = pl.pallas_call(
        functools.partial(_threshold_kernel, k=_K_TOP, iters=46,
                          interp_iters=14),
        grid=(nrt,),
        in_specs=[pl.BlockSpec((rt, nf), lambda r: (r, 0))],
        out_specs=pl.BlockSpec((rt, 1), lambda r: (r, 0)),
        out_shape=jax.ShapeDtypeStruct((n, 1), f32),
        scratch_shapes=[
            pltpu.VMEM((rt, 1), f32),
            pltpu.VMEM((rt, 1), f32),
            pltpu.VMEM((rt, 1), f32),
            pltpu.VMEM((rt, 1), f32),
        ],
    )(pre)

    rd = min(256, n)
    nrd = n // rd
    fd = min(2048, nf)
    nj2 = nf // fd
    acts, recon = pl.pallas_call(
        functools.partial(_decode_kernel, rd=rd),
        grid=(nj2, nrd),
        in_specs=[
            pl.BlockSpec((rd, fd), lambda j, r: (r, j)),
            pl.BlockSpec((d, fd), lambda j, r: (0, j)),
            pl.BlockSpec((rd, 1), lambda j, r: (r, 0)),
        ],
        out_specs=[
            pl.BlockSpec((rd, fd), lambda j, r: (r, j)),
            pl.BlockSpec((n, d), lambda j, r: (0, 0)),
        ],
        out_shape=[
            jax.ShapeDtypeStruct((n, nf), f32),
            jax.ShapeDtypeStruct((n, d), f32),
        ],
    )(pre, W_dec.astype(f32), thr)

    return recon, acts


# speedup vs baseline: 2.1448x; 1.0372x over previous
import functools

import jax
import jax.numpy as jnp
from jax.experimental import pallas as pl
from jax.experimental.pallas import tpu as pltpu

_K_TOP = 32


def _encode_kernel(x_ref, w_ref, b_ref, out_ref):
    acc = jax.lax.dot_general(
        x_ref[...], w_ref[...],
        dimension_numbers=(((1,), (1,)), ((), ())),
        preferred_element_type=jnp.float32,
        precision=jax.lax.Precision.DEFAULT,
    )
    out_ref[...] = acc + b_ref[...]


def _threshold_kernel(p_ref, t_ref, lo_ref, hi_ref, cl_ref, ch_ref,
                      *, k, iters, interp_iters):
    P = p_ref[...]
    kf = jnp.float32(k)
    rows = P.shape[0]

    def count(t):
        return jnp.sum((P >= t).astype(jnp.float32), axis=1, keepdims=True)

    rmax = jnp.max(P, axis=1, keepdims=True)
    rmin = jnp.min(P, axis=1, keepdims=True)
    lo_ref[...] = rmin
    hi_ref[...] = rmax
    cl_ref[...] = jnp.full_like(rmax, jnp.float32(P.shape[1]))
    ch_ref[...] = jnp.full_like(rmax, jnp.float32(1.0))

    def cond(st):
        i, ndone = st
        return jnp.logical_and(i < iters, ndone < rows)

    def body(st):
        i, _ = st
        lo = lo_ref[...]
        hi = hi_ref[...]
        llo = jnp.log(jnp.maximum(cl_ref[...], 1.0))
        lhi = jnp.log(jnp.maximum(ch_ref[...], 0.5))
        lk = jnp.log(kf)
        frac = (llo - lk) / jnp.maximum(llo - lhi, jnp.float32(1e-6))
        frac = jnp.clip(frac, 0.08, 0.92)
        frac = jnp.where(i < interp_iters, frac, jnp.float32(0.5))
        mid = lo + frac * (hi - lo)
        c = count(mid)
        ge = c >= kf
        le = c <= kf
        lo_ref[...] = jnp.where(ge, mid, lo)
        hi_ref[...] = jnp.where(le, mid, hi)
        cl_ref[...] = jnp.where(ge, c, cl_ref[...])
        ch_ref[...] = jnp.where(le, c, ch_ref[...])
        done = jnp.logical_or(c == kf,
                              jnp.logical_or(mid == lo, mid == hi))
        return i + 1, jnp.sum(done.astype(jnp.float32))

    jax.lax.while_loop(cond, body, (jnp.int32(0), jnp.float32(0.0)))
    t_ref[...] = lo_ref[...]


def _decode_kernel(p_ref, w_ref, t_ref, acts_ref, recon_ref, *, rd):
    j = pl.program_id(0)
    r = pl.program_id(1)
    tile = p_ref[...]
    t = t_ref[...]
    acts = jnp.where(tile >= t, jnp.maximum(tile, 0.0), 0.0)
    acts_ref[...] = acts
    contrib = jax.lax.dot_general(
        acts, w_ref[...],
        dimension_numbers=(((1,), (1,)), ((), ())),
        preferred_element_type=jnp.float32,
        precision=jax.lax.Precision.DEFAULT,
    )
    rs = pl.ds(r * rd, rd)

    @pl.when(j == 0)
    def _():
        recon_ref[rs, :] = contrib

    @pl.when(j > 0)
    def _():
        recon_ref[rs, :] = recon_ref[rs, :] + contrib


def kernel(x, W_enc, b_enc, W_dec):
    n, d = x.shape
    nf = W_enc.shape[0]
    f32 = jnp.float32

    fj = min(2048, nf)
    nj1 = nf // fj
    re = min(1024, n)
    nre = n // re
    b2 = b_enc.reshape(1, nf).astype(f32)
    pre = pl.pallas_call(
        _encode_kernel,
        grid=(nj1, nre),
        in_specs=[
            pl.BlockSpec((re, d), lambda j, r: (r, 0)),
            pl.BlockSpec((fj, d), lambda j, r: (j, 0)),
            pl.BlockSpec((1, fj), lambda j, r: (0, j)),
        ],
        out_specs=pl.BlockSpec((re, fj), lambda j, r: (r, j)),
        out_shape=jax.ShapeDtypeStruct((n, nf), f32),
    )(x.astype(f32), W_enc.astype(f32), b2)

    rt = min(128, n)
    nrt = n // rt
    thr = pl.pallas_call(
        functools.partial(_threshold_kernel, k=_K_TOP, iters=46,
                          interp_iters=14),
        grid=(nrt,),
        in_specs=[pl.BlockSpec((rt, nf), lambda r: (r, 0))],
        out_specs=pl.BlockSpec((rt, 1), lambda r: (r, 0)),
        out_shape=jax.ShapeDtypeStruct((n, 1), f32),
        scratch_shapes=[
            pltpu.VMEM((rt, 1), f32),
            pltpu.VMEM((rt, 1), f32),
            pltpu.VMEM((rt, 1), f32),
            pltpu.VMEM((rt, 1), f32),
        ],
    )(pre)

    rd = min(512, n)
    nrd = n // rd
    fd = min(2048, nf)
    nj2 = nf // fd
    acts, recon = pl.pallas_call(
        functools.partial(_decode_kernel, rd=rd),
        grid=(nj2, nrd),
        in_specs=[
            pl.BlockSpec((rd, fd), lambda j, r: (r, j)),
            pl.BlockSpec((d, fd), lambda j, r: (0, j)),
            pl.BlockSpec((rd, 1), lambda j, r: (r, 0)),
        ],
        out_specs=[
            pl.BlockSpec((rd, fd), lambda j, r: (r, j)),
            pl.BlockSpec((n, d), lambda j, r: (0, 0)),
        ],
        out_shape=[
            jax.ShapeDtypeStruct((n, nf), f32),
            jax.ShapeDtypeStruct((n, d), f32),
        ],
    )(pre, W_dec.astype(f32), thr)

    return recon, acts


# speedup vs baseline: 2.2255x; 1.0376x over previous
import functools

import jax
import jax.numpy as jnp
from jax.experimental import pallas as pl
from jax.experimental.pallas import tpu as pltpu

_K_TOP = 32


def _encode_kernel(x_ref, w_ref, b_ref, out_ref):
    acc = jax.lax.dot_general(
        x_ref[...], w_ref[...],
        dimension_numbers=(((1,), (1,)), ((), ())),
        preferred_element_type=jnp.float32,
        precision=jax.lax.Precision.DEFAULT,
    )
    out_ref[...] = acc + b_ref[...]


def _threshold_kernel(p_ref, t_ref, lo_ref, hi_ref, cl_ref, ch_ref,
                      *, k, iters, interp_iters):
    P = p_ref[...]
    kf = jnp.float32(k)
    rows = P.shape[0]

    def count(t):
        return jnp.sum((P >= t).astype(jnp.float32), axis=1, keepdims=True)

    rmax = jnp.max(P, axis=1, keepdims=True)
    rmin = jnp.min(P, axis=1, keepdims=True)
    lo_ref[...] = rmin
    hi_ref[...] = rmax
    cl_ref[...] = jnp.full_like(rmax, jnp.float32(P.shape[1]))
    ch_ref[...] = jnp.full_like(rmax, jnp.float32(1.0))

    def cond(st):
        i, ndone = st
        return jnp.logical_and(i < iters, ndone < rows)

    def body(st):
        i, _ = st
        lo = lo_ref[...]
        hi = hi_ref[...]
        llo = jnp.log(jnp.maximum(cl_ref[...], 1.0))
        lhi = jnp.log(jnp.maximum(ch_ref[...], 0.5))
        lk = jnp.log(kf)
        frac = (llo - lk) / jnp.maximum(llo - lhi, jnp.float32(1e-6))
        frac = jnp.clip(frac, 0.08, 0.92)
        frac = jnp.where(i < interp_iters, frac, jnp.float32(0.5))
        mid = lo + frac * (hi - lo)
        c = count(mid)
        ge = c >= kf
        le = c <= kf
        lo_ref[...] = jnp.where(ge, mid, lo)
        hi_ref[...] = jnp.where(le, mid, hi)
        cl_ref[...] = jnp.where(ge, c, cl_ref[...])
        ch_ref[...] = jnp.where(le, c, ch_ref[...])
        done = jnp.logical_or(c == kf,
                              jnp.logical_or(mid == lo, mid == hi))
        return i + 1, jnp.sum(done.astype(jnp.float32))

    jax.lax.while_loop(cond, body, (jnp.int32(0), jnp.float32(0.0)))
    t_ref[...] = lo_ref[...]


def _decode_kernel(p_ref, w_ref, t_ref, acts_ref, recon_ref, *, rd):
    j = pl.program_id(0)
    r = pl.program_id(1)
    tile = p_ref[...]
    t = t_ref[...]
    acts = jnp.where(tile >= t, jnp.maximum(tile, 0.0), 0.0)
    acts_ref[...] = acts
    contrib = jax.lax.dot_general(
        acts, w_ref[...],
        dimension_numbers=(((1,), (1,)), ((), ())),
        preferred_element_type=jnp.float32,
        precision=jax.lax.Precision.DEFAULT,
    )
    rs = pl.ds(r * rd, rd)

    @pl.when(j == 0)
    def _():
        recon_ref[rs, :] = contrib

    @pl.when(j > 0)
    def _():
        recon_ref[rs, :] = recon_ref[rs, :] + contrib


def kernel(x, W_enc, b_enc, W_dec):
    n, d = x.shape
    nf = W_enc.shape[0]
    f32 = jnp.float32

    fj = min(2048, nf)
    nj1 = nf // fj
    re = min(2048, n)
    nre = n // re
    b2 = b_enc.reshape(1, nf).astype(f32)
    pre = pl.pallas_call(
        _encode_kernel,
        grid=(nj1, nre),
        in_specs=[
            pl.BlockSpec((re, d), lambda j, r: (r, 0)),
            pl.BlockSpec((fj, d), lambda j, r: (j, 0)),
            pl.BlockSpec((1, fj), lambda j, r: (0, j)),
        ],
        out_specs=pl.BlockSpec((re, fj), lambda j, r: (r, j)),
        out_shape=jax.ShapeDtypeStruct((n, nf), f32),
    )(x.astype(f32), W_enc.astype(f32), b2)

    rt = min(128, n)
    nrt = n // rt
    thr = pl.pallas_call(
        functools.partial(_threshold_kernel, k=_K_TOP, iters=46,
                          interp_iters=14),
        grid=(nrt,),
        in_specs=[pl.BlockSpec((rt, nf), lambda r: (r, 0))],
        out_specs=pl.BlockSpec((rt, 1), lambda r: (r, 0)),
        out_shape=jax.ShapeDtypeStruct((n, 1), f32),
        scratch_shapes=[
            pltpu.VMEM((rt, 1), f32),
            pltpu.VMEM((rt, 1), f32),
            pltpu.VMEM((rt, 1), f32),
            pltpu.VMEM((rt, 1), f32),
        ],
    )(pre)

    rd = min(1024, n)
    nrd = n // rd
    fd = min(1024, nf)
    nj2 = nf // fd
    acts, recon = pl.pallas_call(
        functools.partial(_decode_kernel, rd=rd),
        grid=(nj2, nrd),
        in_specs=[
            pl.BlockSpec((rd, fd), lambda j, r: (r, j)),
            pl.BlockSpec((d, fd), lambda j, r: (0, j)),
            pl.BlockSpec((rd, 1), lambda j, r: (r, 0)),
        ],
        out_specs=[
            pl.BlockSpec((rd, fd), lambda j, r: (r, j)),
            pl.BlockSpec((n, d), lambda j, r: (0, 0)),
        ],
        out_shape=[
            jax.ShapeDtypeStruct((n, nf), f32),
            jax.ShapeDtypeStruct((n, d), f32),
        ],
    )(pre, W_dec.astype(f32), thr)

    return recon, acts
